# baseline (speedup 1.0000x reference)
import jax
import jax.numpy as jnp
from jax import lax
from jax.experimental import pallas as pl
from jax.experimental.pallas import tpu as pltpu
from jax.experimental.pallas import tpu_sc as plsc

NC, NS = 2, 16
NW = NC * NS
CHUNK = 128
NBUF = 8
LEAD = 4


def _make_gather(vocab, embed, nchunks):
    mesh = plsc.VectorSubcoreMesh(
        core_axis_name="c", subcore_axis_name="s",
        num_cores=NC, num_subcores=NS)

    def body(table_hbm, idx_hbm, out_hbm, idx_v, bufs, gsems, osems):
        wid = lax.axis_index("s") * NC + lax.axis_index("c")
        pltpu.sync_copy(idx_hbm.at[wid], idx_v)
        row_base = wid * nchunks * CHUNK

        def gather_copy(j, slot):
            return pltpu.make_async_copy(
                table_hbm.at[idx_v.at[j]], bufs.at[slot], gsems.at[slot])

        def out_copy(j, slot):
            return pltpu.make_async_copy(
                bufs.at[slot],
                out_hbm.at[pl.ds(row_base + j * CHUNK, CHUNK)],
                osems.at[slot])

        for b in range(LEAD):
            gather_copy(b, b).start()

        @pl.loop(0, nchunks, step=NBUF)
        def _(j0):
            for b in range(NBUF):
                j = j0 + b
                slot_ahead = (b + LEAD) % NBUF

                @pl.when(j >= LEAD)
                def _():
                    out_copy(j - LEAD, slot_ahead).wait()

                @pl.when(j + LEAD < nchunks)
                def _():
                    gather_copy(j + LEAD, slot_ahead).start()

                gather_copy(j, b).wait()
                out_copy(j, b).start()

        for j in range(nchunks - LEAD, nchunks):
            out_copy(j, j % NBUF).wait()

    return pl.kernel(
        body,
        out_type=jax.ShapeDtypeStruct((NW * nchunks * CHUNK, embed),
                                      jnp.float32),
        mesh=mesh,
        compiler_params=pltpu.CompilerParams(use_tc_tiling_on_sc=False),
        scratch_types=[
            pltpu.VMEM((nchunks, CHUNK), jnp.int32),
            pltpu.VMEM((NBUF, CHUNK, embed), jnp.float32),
            pltpu.SemaphoreType.DMA((NBUF,)),
            pltpu.SemaphoreType.DMA((NBUF,)),
        ],
    )


def kernel(inputs, table):
    b, s = inputs.shape
    vocab, embed = table.shape
    total = b * s
    nchunks = total // (NW * CHUNK)
    idx = inputs.reshape(NW, nchunks, CHUNK).astype(jnp.int32)
    out = _make_gather(vocab, embed, nchunks)(table, idx)
    return out.reshape(b, s, embed)


# speedup vs baseline: 1.2188x; 1.2188x over previous
import jax
import jax.numpy as jnp
from jax import lax
from jax.experimental import pallas as pl
from jax.experimental.pallas import tpu as pltpu
from jax.experimental.pallas import tpu_sc as plsc

NC, NS = 2, 16
NW = NC * NS
CHUNK = 128
NBUF = 4
LEAD = 2


def _make_gather(vocab, embed, nchunks):
    mesh = plsc.VectorSubcoreMesh(
        core_axis_name="c", subcore_axis_name="s",
        num_cores=NC, num_subcores=NS)

    def body(table_hbm, idx_hbm, out_hbm, idx_v, bufs, gsems, osems):
        wid = lax.axis_index("s") * NC + lax.axis_index("c")
        pltpu.sync_copy(idx_hbm.at[wid], idx_v)
        row_base = wid * nchunks * CHUNK

        def gather_copy(j, slot):
            return pltpu.make_async_copy(
                table_hbm.at[idx_v.at[j]], bufs.at[slot], gsems.at[slot])

        def out_copy(j, slot):
            return pltpu.make_async_copy(
                bufs.at[slot],
                out_hbm.at[pl.ds(row_base + j * CHUNK, CHUNK)],
                osems.at[slot])

        for b in range(LEAD):
            gather_copy(b, b).start()

        @pl.loop(0, nchunks, step=NBUF)
        def _(j0):
            for b in range(NBUF):
                j = j0 + b
                slot_ahead = (b + LEAD) % NBUF

                @pl.when(j >= LEAD)
                def _():
                    out_copy(j - LEAD, slot_ahead).wait()

                @pl.when(j + LEAD < nchunks)
                def _():
                    gather_copy(j + LEAD, slot_ahead).start()

                gather_copy(j, b).wait()
                out_copy(j, b).start()

        for j in range(nchunks - LEAD, nchunks):
            out_copy(j, j % NBUF).wait()

    return pl.kernel(
        body,
        out_type=jax.ShapeDtypeStruct((NW * nchunks * CHUNK, embed),
                                      jnp.float32),
        mesh=mesh,
        compiler_params=pltpu.CompilerParams(use_tc_tiling_on_sc=True),
        scratch_types=[
            pltpu.VMEM((nchunks, CHUNK), jnp.int32),
            pltpu.VMEM((NBUF, CHUNK, embed), jnp.float32),
            pltpu.SemaphoreType.DMA((NBUF,)),
            pltpu.SemaphoreType.DMA((NBUF,)),
        ],
    )


def kernel(inputs, table):
    b, s = inputs.shape
    vocab, embed = table.shape
    total = b * s
    nchunks = total // (NW * CHUNK)
    idx = inputs.reshape(NW, nchunks, CHUNK).astype(jnp.int32)
    table_p = jnp.pad(table, ((0, 0), (0, 128 - embed)))
    out = _make_gather(vocab, 128, nchunks)(table_p, idx)
    return out[:, :embed].reshape(b, s, embed)
